# Initial kernel scaffold; baseline (speedup 1.0000x reference)
import jax
import jax.numpy as jnp
from jax.experimental import pallas as pl


def kernel(x, edge_index, W1, b1, W2, b2):
    raise NotImplementedError("write your pallas kernel here")


# speedup vs baseline: 2.2847x; 2.2847x over previous
import functools
import jax
import jax.numpy as jnp
from jax import lax
from jax.experimental import pallas as pl
from jax.experimental.pallas import tpu as pltpu

N_NODES = 10000
D = 256
BLK_M = 1000


def _mm1_body(x_ref, w1_ref, b1_ref, deg_ref, h_ref):
    acc = lax.dot_general(x_ref[...], w1_ref[...], (((1,), (1,)), ((), ())),
                          preferred_element_type=jnp.float32)
    acc = jnp.maximum(acc + b1_ref[...], 0.0)
    dinv = lax.rsqrt(jnp.maximum(deg_ref[...], 1.0))
    h_ref[...] = acc * dinv


def _mm2_body(agg_ref, w2_ref, b2_ref, deg_ref, out_ref):
    dinv = lax.rsqrt(jnp.maximum(deg_ref[...], 1.0))
    a = agg_ref[...] * dinv
    acc = lax.dot_general(a, w2_ref[...], (((1,), (1,)), ((), ())),
                          preferred_element_type=jnp.float32)
    out_ref[...] = acc + b2_ref[...]


def _mm1(x, W1, b1, deg):
    grid = N_NODES // BLK_M
    return pl.pallas_call(
        _mm1_body,
        grid=(grid,),
        in_specs=[
            pl.BlockSpec((BLK_M, D), lambda i: (i, 0)),
            pl.BlockSpec((D, D), lambda i: (0, 0)),
            pl.BlockSpec((1, D), lambda i: (0, 0)),
            pl.BlockSpec((BLK_M, 1), lambda i: (i, 0)),
        ],
        out_specs=pl.BlockSpec((BLK_M, D), lambda i: (i, 0)),
        out_shape=jax.ShapeDtypeStruct((N_NODES, D), jnp.float32),
    )(x, W1, b1, deg)


def _mm2(agg, W2, b2, deg):
    grid = N_NODES // BLK_M
    return pl.pallas_call(
        _mm2_body,
        grid=(grid,),
        in_specs=[
            pl.BlockSpec((BLK_M, D), lambda i: (i, 0)),
            pl.BlockSpec((D, D), lambda i: (0, 0)),
            pl.BlockSpec((1, D), lambda i: (0, 0)),
            pl.BlockSpec((BLK_M, 1), lambda i: (i, 0)),
        ],
        out_specs=pl.BlockSpec((BLK_M, D), lambda i: (i, 0)),
        out_shape=jax.ShapeDtypeStruct((N_NODES, D), jnp.float32),
    )(agg, W2, b2, deg)


@jax.jit
def _run(x, edge_index, W1, b1, W2, b2):
    edge_index = edge_index.astype(jnp.int32)
    row, col = edge_index[0], edge_index[1]
    E = row.shape[0]
    ones = jnp.ones((E,), dtype=jnp.float32)
    deg = jnp.zeros((N_NODES,), jnp.float32).at[row].add(ones).at[col].add(ones)
    deg = deg[:, None]

    h = _mm1(x, W1, b1[None, :], deg)
    agg = jnp.zeros((N_NODES, D), jnp.float32).at[col].add(h[row])
    return _mm2(agg, W2, b2[None, :], deg)


def kernel(x, edge_index, W1, b1, W2, b2):
    return _run(x, edge_index, W1, b1, W2, b2)


# speedup vs baseline: 17.6944x; 7.7447x over previous
import functools
import jax
import jax.numpy as jnp
from jax import lax
from jax.experimental import pallas as pl
from jax.experimental.pallas import tpu as pltpu
from jax.experimental.pallas import tpu_sc as plsc

N = 10000
NPAD = 10240
E = 160000
D = 256
H = 128
NS = 16
NC = 2
K = 100
NCH = (E // NS) // K
DEG_NCH = (2 * E // (NS * NC)) // K
BLK_M = 1000


def _deg_body(catidx, deg_out, idx_v, ones_v, zbuf, shared, sem):
    c = lax.axis_index("c")
    s = lax.axis_index("s")
    w = s * NC + c

    pltpu.sync_copy(catidx.at[w], idx_v)

    one = jnp.ones((16,), jnp.float32)
    for j in range(7):
        ones_v[pl.ds(j * 16, 16)] = one
    zero = jnp.zeros((16,), jnp.float32)

    @pl.loop(0, 40)
    def _(i):
        zbuf[pl.ds(i * 16, 16)] = zero

    pltpu.sync_copy(zbuf, shared.at[pl.ds(s * 640, 640)])
    plsc.subcore_barrier()

    GRP = 10

    @pl.loop(0, DEG_NCH // GRP)
    def _(g):
        for j in range(GRP):
            pltpu.async_copy(
                ones_v.at[pl.ds(0, K)],
                shared.at[idx_v.at[g * GRP + j]],
                sem, add=True)
        for j in range(GRP):
            pltpu.make_async_copy(
                ones_v.at[pl.ds(0, K)],
                shared.at[idx_v.at[g * GRP + j]],
                sem).wait()

    plsc.subcore_barrier()
    pltpu.sync_copy(shared.at[pl.ds(s * 640, 640)],
                    deg_out.at[pl.ds(c * NPAD + s * 640, 640)])


@functools.partial(
    pl.kernel,
    out_type=jax.ShapeDtypeStruct((NC * NPAD,), jnp.float32),
    mesh=plsc.VectorSubcoreMesh(core_axis_name="c", subcore_axis_name="s"),
    scratch_types=[
        pltpu.VMEM((DEG_NCH, K), jnp.int32),
        pltpu.VMEM((112,), jnp.float32),
        pltpu.VMEM((640,), jnp.float32),
        pltpu.VMEM_SHARED((NPAD,), jnp.float32),
        pltpu.SemaphoreType.DMA,
    ],
)
def _deg_kernel(catidx, deg_out, idx_v, ones_v, zbuf, shared, sem):
    _deg_body(catidx, deg_out, idx_v, ones_v, zbuf, shared, sem)


GRP = 5
NG = NCH // (2 * GRP)


def _agg_group(h_ref, shared, buf, sems, rows_cur, cols_cur, rows_next,
               par0, guard_last):
    for jj in range(GRP):
        bp = (par0 + jj) % 2
        nparity = (par0 + jj + 1) % 2
        if jj < GRP - 1:
            pltpu.async_copy(h_ref.at[rows_cur.at[jj + 1]],
                             buf.at[nparity], sems[nparity])
        elif guard_last is None:
            pltpu.async_copy(h_ref.at[rows_next.at[0]],
                             buf.at[nparity], sems[nparity])
        else:
            @pl.when(guard_last)
            def _():
                pltpu.async_copy(h_ref.at[rows_next.at[0]],
                                 buf.at[nparity], sems[nparity])
        pltpu.make_async_copy(h_ref.at[rows_cur.at[jj]],
                              buf.at[bp], sems[bp]).wait()
        pltpu.sync_copy(buf.at[bp], shared.at[cols_cur.at[jj]], add=True)


def _agg_half(s, h_ref, o_ref, rowi, coli, ri0, ri1, ci0, ci1,
              buf, shared, sem0, sem1):
    sems = (sem0, sem1)
    pltpu.sync_copy(rowi.at[s, 0], ri0)
    pltpu.sync_copy(coli.at[s, 0], ci0)
    pltpu.async_copy(h_ref.at[ri0.at[0]], buf.at[0], sem0)

    @pl.loop(0, NG)
    def _(g):
        pltpu.sync_copy(rowi.at[s, 2 * g + 1], ri1)
        pltpu.sync_copy(coli.at[s, 2 * g + 1], ci1)
        _agg_group(h_ref, shared, buf, sems, ri0, ci0, ri1, 0, None)
        nxt = jnp.minimum(2 * g + 2, 2 * NG - 1)

        @pl.when(g < NG - 1)
        def _():
            pltpu.sync_copy(rowi.at[s, nxt], ri0)
            pltpu.sync_copy(coli.at[s, nxt], ci0)

        _agg_group(h_ref, shared, buf, sems, ri1, ci1, ri0, GRP % 2,
                   g < NG - 1)

    plsc.subcore_barrier()
    for k in range(5):
        pltpu.sync_copy(shared.at[pl.ds(s * 640 + k * 128, 128)],
                        o_ref.at[pl.ds(s * 640 + k * 128, 128)])


def _agg_body(h0, h1, rowi, coli, o0, o1,
              ri0, ri1, ci0, ci1, buf, shared, sem0, sem1):
    c = lax.axis_index("c")
    s = lax.axis_index("s")

    zero = jnp.zeros((16,), jnp.float32)

    @pl.loop(0, 512)
    def _(i):
        buf[0, i // 8, pl.ds((i % 8) * 16, 16)] = zero

    for k in range(10):
        pltpu.sync_copy(buf.at[0, pl.ds(0, 64)],
                        shared.at[pl.ds(s * 640 + k * 64, 64)])
    plsc.subcore_barrier()

    @pl.when(c == 0)
    def _():
        _agg_half(s, h0, o0, rowi, coli, ri0, ri1, ci0, ci1,
                  buf, shared, sem0, sem1)

    @pl.when(c == 1)
    def _():
        _agg_half(s, h1, o1, rowi, coli, ri0, ri1, ci0, ci1,
                  buf, shared, sem0, sem1)


@functools.partial(
    pl.kernel,
    out_type=(jax.ShapeDtypeStruct((NPAD, H), jnp.float32),
              jax.ShapeDtypeStruct((NPAD, H), jnp.float32)),
    mesh=plsc.VectorSubcoreMesh(core_axis_name="c", subcore_axis_name="s"),
    scratch_types=[
        pltpu.VMEM((GRP, K), jnp.int32),
        pltpu.VMEM((GRP, K), jnp.int32),
        pltpu.VMEM((GRP, K), jnp.int32),
        pltpu.VMEM((GRP, K), jnp.int32),
        pltpu.VMEM((2, K, H), jnp.float32),
        pltpu.VMEM_SHARED((NPAD, H), jnp.float32),
        pltpu.SemaphoreType.DMA,
        pltpu.SemaphoreType.DMA,
    ],
)
def _agg_kernel(h0, h1, rowi, coli, o0, o1,
                ri0, ri1, ci0, ci1, buf, shared, sem0, sem1):
    _agg_body(h0, h1, rowi, coli, o0, o1,
              ri0, ri1, ci0, ci1, buf, shared, sem0, sem1)


def _mm1_body(x_ref, w1_ref, b1_ref, deg_ref, h_ref):
    acc = lax.dot_general(x_ref[...], w1_ref[...], (((1,), (1,)), ((), ())),
                          preferred_element_type=jnp.float32)
    h = jnp.maximum(acc + b1_ref[...], 0.0)
    d = deg_ref[...]
    dinv = lax.rsqrt(jnp.maximum(d[:, 0:1] + d[:, 1:2], 1.0))
    h_ref[0] = h * dinv


def _mm1(x, W1, b1, degT):
    return pl.pallas_call(
        _mm1_body,
        grid=(N // BLK_M, NC),
        in_specs=[
            pl.BlockSpec((BLK_M, D), lambda i, j: (i, 0)),
            pl.BlockSpec((H, D), lambda i, j: (j, 0)),
            pl.BlockSpec((1, H), lambda i, j: (0, j)),
            pl.BlockSpec((BLK_M, 2), lambda i, j: (i, 0)),
        ],
        out_specs=pl.BlockSpec((1, BLK_M, H), lambda i, j: (j, i, 0)),
        out_shape=jax.ShapeDtypeStruct((NC, N, H), jnp.float32),
    )(x, W1, b1, degT)


def _mm2_body(a0_ref, a1_ref, w2a_ref, w2b_ref, b2_ref, deg_ref, out_ref):
    d = deg_ref[...]
    dinv = lax.rsqrt(jnp.maximum(d[:, 0:1] + d[:, 1:2], 1.0))
    acc = lax.dot_general(a0_ref[...] * dinv, w2a_ref[...],
                          (((1,), (1,)), ((), ())),
                          preferred_element_type=jnp.float32)
    acc = acc + lax.dot_general(a1_ref[...] * dinv, w2b_ref[...],
                                (((1,), (1,)), ((), ())),
                                preferred_element_type=jnp.float32)
    out_ref[...] = acc + b2_ref[...]


def _mm2(a0, a1, W2, b2, degT):
    return pl.pallas_call(
        _mm2_body,
        grid=(N // BLK_M,),
        in_specs=[
            pl.BlockSpec((BLK_M, H), lambda i: (i, 0)),
            pl.BlockSpec((BLK_M, H), lambda i: (i, 0)),
            pl.BlockSpec((D, H), lambda i: (0, 0)),
            pl.BlockSpec((D, H), lambda i: (0, 1)),
            pl.BlockSpec((1, D), lambda i: (0, 0)),
            pl.BlockSpec((BLK_M, 2), lambda i: (i, 0)),
        ],
        out_specs=pl.BlockSpec((BLK_M, D), lambda i: (i, 0)),
        out_shape=jax.ShapeDtypeStruct((N, D), jnp.float32),
    )(a0, a1, W2, W2, b2, degT)


@jax.jit
def _run(x, edge_index, W1, b1, W2, b2):
    edge32 = edge_index.astype(jnp.int32)
    row, col = edge32[0], edge32[1]
    catidx = jnp.concatenate([row, col]).reshape(NC * NS, DEG_NCH, K)
    rowi = row.reshape(NS, NCH // GRP, GRP, K)
    coli = col.reshape(NS, NCH // GRP, GRP, K)

    deg2 = _deg_kernel(catidx)
    degT = deg2.reshape(NC, NPAD).T

    h2 = _mm1(x, W1, b1[None, :], degT)
    a0, a1 = _agg_kernel(h2[0], h2[1], rowi, coli)
    return _mm2(a0, a1, W2, b2[None, :], degT)


def kernel(x, edge_index, W1, b1, W2, b2):
    return _run(x, edge_index, W1, b1, W2, b2)


# speedup vs baseline: 18.9660x; 1.0719x over previous
import functools
import jax
import jax.numpy as jnp
from jax import lax
from jax.experimental import pallas as pl
from jax.experimental.pallas import tpu as pltpu
from jax.experimental.pallas import tpu_sc as plsc

N = 10000
NPAD = 10240
E = 160000
D = 256
H = 128
NS = 16
NC = 2
K = 125
NCH = (E // NS) // K
GRP = 5
NG = NCH // (2 * GRP)
BLK_M = 1000


def _deg_scatter(idx_v, ones_v, shared, sem):
    @pl.loop(0, NCH // GRP)
    def _(g):
        for j in range(GRP):
            pltpu.async_copy(
                ones_v.at[pl.ds(0, K)],
                shared.at[idx_v.at[g, j]],
                sem, add=True)
        for j in range(GRP):
            pltpu.make_async_copy(
                ones_v.at[pl.ds(0, K)],
                shared.at[idx_v.at[g, j]],
                sem).wait()


def _deg_body(rowi, coli, deg_out, idx_v, ones_v, zbuf, shared, sem):
    c = lax.axis_index("c")
    s = lax.axis_index("s")

    @pl.when(c == 0)
    def _():
        pltpu.sync_copy(rowi.at[s], idx_v)

    @pl.when(c == 1)
    def _():
        pltpu.sync_copy(coli.at[s], idx_v)

    one = jnp.ones((16,), jnp.float32)
    for j in range(8):
        ones_v[pl.ds(j * 16, 16)] = one
    zero = jnp.zeros((16,), jnp.float32)

    @pl.loop(0, 40)
    def _(i):
        zbuf[pl.ds(i * 16, 16)] = zero

    pltpu.sync_copy(zbuf, shared.at[pl.ds(s * 640, 640)])
    plsc.subcore_barrier()
    _deg_scatter(idx_v, ones_v, shared, sem)
    plsc.subcore_barrier()
    pltpu.sync_copy(shared.at[pl.ds(s * 640, 640)],
                    deg_out.at[pl.ds(c * NPAD + s * 640, 640)])


@functools.partial(
    pl.kernel,
    out_type=jax.ShapeDtypeStruct((NC * NPAD,), jnp.float32),
    mesh=plsc.VectorSubcoreMesh(core_axis_name="c", subcore_axis_name="s"),
    scratch_types=[
        pltpu.VMEM((NCH // GRP, GRP, K), jnp.int32),
        pltpu.VMEM((128,), jnp.float32),
        pltpu.VMEM((640,), jnp.float32),
        pltpu.VMEM_SHARED((NPAD,), jnp.float32),
        pltpu.SemaphoreType.DMA,
    ],
)
def _deg_kernel(rowi, coli, deg_out, idx_v, ones_v, zbuf, shared, sem):
    _deg_body(rowi, coli, deg_out, idx_v, ones_v, zbuf, shared, sem)


def _agg_group(h_ref, shared, buf, sems, rows_cur, cols_cur, rows_next,
               par0, guard_last):
    for jj in range(GRP):
        bp = (par0 + jj) % 2
        nparity = (par0 + jj + 1) % 2
        if jj < GRP - 1:
            pltpu.async_copy(h_ref.at[rows_cur.at[jj + 1]],
                             buf.at[nparity], sems[nparity])
        elif guard_last is None:
            pltpu.async_copy(h_ref.at[rows_next.at[0]],
                             buf.at[nparity], sems[nparity])
        else:
            @pl.when(guard_last)
            def _():
                pltpu.async_copy(h_ref.at[rows_next.at[0]],
                                 buf.at[nparity], sems[nparity])
        pltpu.make_async_copy(h_ref.at[rows_cur.at[jj]],
                              buf.at[bp], sems[bp]).wait()
        pltpu.sync_copy(buf.at[bp], shared.at[cols_cur.at[jj]], add=True)


def _agg_half(s, h_ref, o_ref, rowi, coli, ri0, ri1, ci0, ci1,
              buf, shared, sem0, sem1):
    sems = (sem0, sem1)
    pltpu.sync_copy(rowi.at[s, 0], ri0)
    pltpu.sync_copy(coli.at[s, 0], ci0)
    pltpu.async_copy(h_ref.at[ri0.at[0]], buf.at[0], sem0)

    @pl.loop(0, NG)
    def _(g):
        pltpu.sync_copy(rowi.at[s, 2 * g + 1], ri1)
        pltpu.sync_copy(coli.at[s, 2 * g + 1], ci1)
        _agg_group(h_ref, shared, buf, sems, ri0, ci0, ri1, 0, None)
        nxt = jnp.minimum(2 * g + 2, 2 * NG - 1)

        @pl.when(g < NG - 1)
        def _():
            pltpu.sync_copy(rowi.at[s, nxt], ri0)
            pltpu.sync_copy(coli.at[s, nxt], ci0)

        _agg_group(h_ref, shared, buf, sems, ri1, ci1, ri0, GRP % 2,
                   g < NG - 1)

    plsc.subcore_barrier()
    for k in range(5):
        pltpu.sync_copy(shared.at[pl.ds(s * 640 + k * 128, 128)],
                        o_ref.at[pl.ds(s * 640 + k * 128, 128)])


def _agg_body(h2, rowi, coli, o0, o1,
              ri0, ri1, ci0, ci1, buf, shared, sem0, sem1):
    c = lax.axis_index("c")
    s = lax.axis_index("s")

    zero = jnp.zeros((16,), jnp.float32)

    @pl.loop(0, 512)
    def _(i):
        buf[0, i // 8, pl.ds((i % 8) * 16, 16)] = zero

    for k in range(10):
        pltpu.sync_copy(buf.at[0, pl.ds(0, 64)],
                        shared.at[pl.ds(s * 640 + k * 64, 64)])
    plsc.subcore_barrier()

    @pl.when(c == 0)
    def _():
        _agg_half(s, h2.at[0], o0, rowi, coli, ri0, ri1, ci0, ci1,
                  buf, shared, sem0, sem1)

    @pl.when(c == 1)
    def _():
        _agg_half(s, h2.at[1], o1, rowi, coli, ri0, ri1, ci0, ci1,
                  buf, shared, sem0, sem1)


@functools.partial(
    pl.kernel,
    out_type=(jax.ShapeDtypeStruct((NPAD, H), jnp.float32),
              jax.ShapeDtypeStruct((NPAD, H), jnp.float32)),
    mesh=plsc.VectorSubcoreMesh(core_axis_name="c", subcore_axis_name="s"),
    scratch_types=[
        pltpu.VMEM((GRP, K), jnp.int32),
        pltpu.VMEM((GRP, K), jnp.int32),
        pltpu.VMEM((GRP, K), jnp.int32),
        pltpu.VMEM((GRP, K), jnp.int32),
        pltpu.VMEM((2, K, H), jnp.float32),
        pltpu.VMEM_SHARED((NPAD, H), jnp.float32),
        pltpu.SemaphoreType.DMA,
        pltpu.SemaphoreType.DMA,
    ],
)
def _agg_kernel(h2, rowi, coli, o0, o1,
                ri0, ri1, ci0, ci1, buf, shared, sem0, sem1):
    _agg_body(h2, rowi, coli, o0, o1,
              ri0, ri1, ci0, ci1, buf, shared, sem0, sem1)


def _mm1_body(x_ref, w1_ref, b1_ref, deg_ref, h_ref):
    acc = lax.dot_general(x_ref[...], w1_ref[...], (((1,), (1,)), ((), ())),
                          preferred_element_type=jnp.float32)
    h = jnp.maximum(acc + b1_ref[...], 0.0)
    d = deg_ref[...]
    dinv = lax.rsqrt(jnp.maximum(d[:, 0:1] + d[:, 1:2], 1.0))
    h_ref[0] = h * dinv


def _mm1(x, W1, b1, degT):
    return pl.pallas_call(
        _mm1_body,
        grid=(N // BLK_M, NC),
        in_specs=[
            pl.BlockSpec((BLK_M, D), lambda i, j: (i, 0)),
            pl.BlockSpec((H, D), lambda i, j: (j, 0)),
            pl.BlockSpec((1, H), lambda i, j: (0, j)),
            pl.BlockSpec((BLK_M, 2), lambda i, j: (i, 0)),
        ],
        out_specs=pl.BlockSpec((1, BLK_M, H), lambda i, j: (j, i, 0)),
        out_shape=jax.ShapeDtypeStruct((NC, N, H), jnp.float32),
    )(x, W1, b1, degT)


def _mm2_body(a0_ref, a1_ref, w2a_ref, w2b_ref, b2_ref, deg_ref, out_ref):
    d = deg_ref[...]
    dinv = lax.rsqrt(jnp.maximum(d[:, 0:1] + d[:, 1:2], 1.0))
    acc = lax.dot_general(a0_ref[...] * dinv, w2a_ref[...],
                          (((1,), (1,)), ((), ())),
                          preferred_element_type=jnp.float32)
    acc = acc + lax.dot_general(a1_ref[...] * dinv, w2b_ref[...],
                                (((1,), (1,)), ((), ())),
                                preferred_element_type=jnp.float32)
    out_ref[...] = acc + b2_ref[...]


def _mm2(a0, a1, W2, b2, degT):
    return pl.pallas_call(
        _mm2_body,
        grid=(N // BLK_M,),
        in_specs=[
            pl.BlockSpec((BLK_M, H), lambda i: (i, 0)),
            pl.BlockSpec((BLK_M, H), lambda i: (i, 0)),
            pl.BlockSpec((D, H), lambda i: (0, 0)),
            pl.BlockSpec((D, H), lambda i: (0, 1)),
            pl.BlockSpec((1, D), lambda i: (0, 0)),
            pl.BlockSpec((BLK_M, 2), lambda i: (i, 0)),
        ],
        out_specs=pl.BlockSpec((BLK_M, D), lambda i: (i, 0)),
        out_shape=jax.ShapeDtypeStruct((N, D), jnp.float32),
    )(a0, a1, W2, W2, b2, degT)


@jax.jit
def _run(x, edge_index, W1, b1, W2, b2):
    edge32 = edge_index.astype(jnp.int32)
    row, col = edge32[0], edge32[1]
    rowi = row.reshape(NS, NCH // GRP, GRP, K)
    coli = col.reshape(NS, NCH // GRP, GRP, K)

    deg2 = _deg_kernel(rowi, coli)
    degT = deg2.reshape(NC, NPAD).T

    h2 = _mm1(x, W1, b1[None, :], degT)
    a0, a1 = _agg_kernel(h2, rowi, coli)
    return _mm2(a0, a1, W2, b2[None, :], degT)


def kernel(x, edge_index, W1, b1, W2, b2):
    return _run(x, edge_index, W1, b1, W2, b2)


# speedup vs baseline: 20.3959x; 1.0754x over previous
import functools
import jax
import jax.numpy as jnp
from jax import lax
from jax.experimental import pallas as pl
from jax.experimental.pallas import tpu as pltpu
from jax.experimental.pallas import tpu_sc as plsc

N = 10000
NPAD = 10240
E = 160000
D = 256
H = 128
NS = 16
NC = 2
K = 125
NCH = (E // NS) // K
GRP = 5
NG = NCH // (2 * GRP)
BLK_M = 2000


def _deg_scatter(idx_v, ones_v, shared, sem):
    @pl.loop(0, NCH // GRP)
    def _(g):
        for j in range(GRP):
            pltpu.async_copy(
                ones_v.at[pl.ds(0, K)],
                shared.at[idx_v.at[g, j]],
                sem, add=True)
        for j in range(GRP):
            pltpu.make_async_copy(
                ones_v.at[pl.ds(0, K)],
                shared.at[idx_v.at[g, j]],
                sem).wait()


def _deg_body(rowi, coli, deg_out, idx_v, ones_v, zbuf, shared, sem):
    c = lax.axis_index("c")
    s = lax.axis_index("s")

    @pl.when(c == 0)
    def _():
        pltpu.sync_copy(rowi.at[s], idx_v)

    @pl.when(c == 1)
    def _():
        pltpu.sync_copy(coli.at[s], idx_v)

    one = jnp.ones((16,), jnp.float32)
    for j in range(8):
        ones_v[pl.ds(j * 16, 16)] = one
    zero = jnp.zeros((16,), jnp.float32)

    @pl.loop(0, 40)
    def _(i):
        zbuf[pl.ds(i * 16, 16)] = zero

    pltpu.sync_copy(zbuf, shared.at[pl.ds(s * 640, 640)])
    plsc.subcore_barrier()
    _deg_scatter(idx_v, ones_v, shared, sem)
    plsc.subcore_barrier()
    pltpu.sync_copy(shared.at[pl.ds(s * 640, 640)],
                    deg_out.at[pl.ds(c * NPAD + s * 640, 640)])


@functools.partial(
    pl.kernel,
    out_type=jax.ShapeDtypeStruct((NC * NPAD,), jnp.float32),
    mesh=plsc.VectorSubcoreMesh(core_axis_name="c", subcore_axis_name="s"),
    scratch_types=[
        pltpu.VMEM((NCH // GRP, GRP, K), jnp.int32),
        pltpu.VMEM((128,), jnp.float32),
        pltpu.VMEM((640,), jnp.float32),
        pltpu.VMEM_SHARED((NPAD,), jnp.float32),
        pltpu.SemaphoreType.DMA,
    ],
)
def _deg_kernel(rowi, coli, deg_out, idx_v, ones_v, zbuf, shared, sem):
    _deg_body(rowi, coli, deg_out, idx_v, ones_v, zbuf, shared, sem)


def _agg_group(h_ref, shared, buf, sems, rows_cur, cols_cur, rows_next,
               par0, guard_last):
    for jj in range(GRP):
        bp = (par0 + jj) % 2
        nparity = (par0 + jj + 1) % 2
        if jj < GRP - 1:
            pltpu.async_copy(h_ref.at[rows_cur.at[jj + 1]],
                             buf.at[nparity], sems[nparity])
        elif guard_last is None:
            pltpu.async_copy(h_ref.at[rows_next.at[0]],
                             buf.at[nparity], sems[nparity])
        else:
            @pl.when(guard_last)
            def _():
                pltpu.async_copy(h_ref.at[rows_next.at[0]],
                                 buf.at[nparity], sems[nparity])
        pltpu.make_async_copy(h_ref.at[rows_cur.at[jj]],
                              buf.at[bp], sems[bp]).wait()
        pltpu.sync_copy(buf.at[bp], shared.at[cols_cur.at[jj]], add=True)


def _agg_half(s, h_ref, o_ref, rowi, coli, ri0, ri1, ci0, ci1,
              buf, shared, sem0, sem1):
    sems = (sem0, sem1)
    pltpu.sync_copy(rowi.at[s, 0], ri0)
    pltpu.sync_copy(coli.at[s, 0], ci0)
    pltpu.async_copy(h_ref.at[ri0.at[0]], buf.at[0], sem0)

    @pl.loop(0, NG)
    def _(g):
        pltpu.sync_copy(rowi.at[s, 2 * g + 1], ri1)
        pltpu.sync_copy(coli.at[s, 2 * g + 1], ci1)
        _agg_group(h_ref, shared, buf, sems, ri0, ci0, ri1, 0, None)
        nxt = jnp.minimum(2 * g + 2, 2 * NG - 1)

        @pl.when(g < NG - 1)
        def _():
            pltpu.sync_copy(rowi.at[s, nxt], ri0)
            pltpu.sync_copy(coli.at[s, nxt], ci0)

        _agg_group(h_ref, shared, buf, sems, ri1, ci1, ri0, GRP % 2,
                   g < NG - 1)

    plsc.subcore_barrier()
    for k in range(5):
        pltpu.sync_copy(shared.at[pl.ds(s * 640 + k * 128, 128)],
                        o_ref.at[pl.ds(s * 640 + k * 128, 128)])


def _agg_body(h2, rowi, coli, o0, o1,
              ri0, ri1, ci0, ci1, buf, shared, sem0, sem1):
    c = lax.axis_index("c")
    s = lax.axis_index("s")

    zero = jnp.zeros((16,), jnp.float32)

    @pl.loop(0, 512)
    def _(i):
        buf[0, i // 8, pl.ds((i % 8) * 16, 16)] = zero

    for k in range(10):
        pltpu.sync_copy(buf.at[0, pl.ds(0, 64)],
                        shared.at[pl.ds(s * 640 + k * 64, 64)])
    plsc.subcore_barrier()

    @pl.when(c == 0)
    def _():
        _agg_half(s, h2.at[0], o0, rowi, coli, ri0, ri1, ci0, ci1,
                  buf, shared, sem0, sem1)

    @pl.when(c == 1)
    def _():
        _agg_half(s, h2.at[1], o1, rowi, coli, ri0, ri1, ci0, ci1,
                  buf, shared, sem0, sem1)


@functools.partial(
    pl.kernel,
    out_type=(jax.ShapeDtypeStruct((NPAD, H), jnp.float32),
              jax.ShapeDtypeStruct((NPAD, H), jnp.float32)),
    mesh=plsc.VectorSubcoreMesh(core_axis_name="c", subcore_axis_name="s"),
    scratch_types=[
        pltpu.VMEM((GRP, K), jnp.int32),
        pltpu.VMEM((GRP, K), jnp.int32),
        pltpu.VMEM((GRP, K), jnp.int32),
        pltpu.VMEM((GRP, K), jnp.int32),
        pltpu.VMEM((2, K, H), jnp.float32),
        pltpu.VMEM_SHARED((NPAD, H), jnp.float32),
        pltpu.SemaphoreType.DMA,
        pltpu.SemaphoreType.DMA,
    ],
)
def _agg_kernel(h2, rowi, coli, o0, o1,
                ri0, ri1, ci0, ci1, buf, shared, sem0, sem1):
    _agg_body(h2, rowi, coli, o0, o1,
              ri0, ri1, ci0, ci1, buf, shared, sem0, sem1)


def _mm1_body(x_ref, w1_ref, b1_ref, deg_ref, h_ref):
    xb = x_ref[...].astype(jnp.bfloat16)
    wb = w1_ref[...].astype(jnp.bfloat16)
    acc = lax.dot_general(xb, wb, (((1,), (1,)), ((), ())),
                          preferred_element_type=jnp.float32)
    h = jnp.maximum(acc + b1_ref[...], 0.0)
    d = deg_ref[...]
    dinv = lax.rsqrt(jnp.maximum(d[:, 0:1] + d[:, 1:2], 1.0))
    h = h * dinv
    h_ref[0] = h[:, :H]
    h_ref[1] = h[:, H:]


def _mm1(x, W1, b1, degT):
    return pl.pallas_call(
        _mm1_body,
        grid=(N // BLK_M,),
        in_specs=[
            pl.BlockSpec((BLK_M, D), lambda i: (i, 0)),
            pl.BlockSpec((D, D), lambda i: (0, 0)),
            pl.BlockSpec((1, D), lambda i: (0, 0)),
            pl.BlockSpec((BLK_M, 2), lambda i: (i, 0)),
        ],
        out_specs=pl.BlockSpec((NC, BLK_M, H), lambda i: (0, i, 0)),
        out_shape=jax.ShapeDtypeStruct((NC, N, H), jnp.float32),
    )(x, W1, b1, degT)


def _mm2_body(a0_ref, a1_ref, w2a_ref, w2b_ref, b2_ref, deg_ref, out_ref):
    d = deg_ref[...]
    dinv = lax.rsqrt(jnp.maximum(d[:, 0:1] + d[:, 1:2], 1.0))
    acc = lax.dot_general((a0_ref[...] * dinv).astype(jnp.bfloat16),
                          w2a_ref[...].astype(jnp.bfloat16),
                          (((1,), (1,)), ((), ())),
                          preferred_element_type=jnp.float32)
    acc = acc + lax.dot_general((a1_ref[...] * dinv).astype(jnp.bfloat16),
                                w2b_ref[...].astype(jnp.bfloat16),
                                (((1,), (1,)), ((), ())),
                                preferred_element_type=jnp.float32)
    out_ref[...] = acc + b2_ref[...]


def _mm2(a0, a1, W2, b2, degT):
    return pl.pallas_call(
        _mm2_body,
        grid=(N // BLK_M,),
        in_specs=[
            pl.BlockSpec((BLK_M, H), lambda i: (i, 0)),
            pl.BlockSpec((BLK_M, H), lambda i: (i, 0)),
            pl.BlockSpec((D, H), lambda i: (0, 0)),
            pl.BlockSpec((D, H), lambda i: (0, 1)),
            pl.BlockSpec((1, D), lambda i: (0, 0)),
            pl.BlockSpec((BLK_M, 2), lambda i: (i, 0)),
        ],
        out_specs=pl.BlockSpec((BLK_M, D), lambda i: (i, 0)),
        out_shape=jax.ShapeDtypeStruct((N, D), jnp.float32),
    )(a0, a1, W2, W2, b2, degT)


@jax.jit
def _run(x, edge_index, W1, b1, W2, b2):
    edge32 = edge_index.astype(jnp.int32)
    row, col = edge32[0], edge32[1]
    rowi = row.reshape(NS, NCH // GRP, GRP, K)
    coli = col.reshape(NS, NCH // GRP, GRP, K)

    deg2 = _deg_kernel(rowi, coli)
    degT = deg2.reshape(NC, NPAD).T

    h2 = _mm1(x, W1, b1[None, :], degT)
    a0, a1 = _agg_kernel(h2, rowi, coli)
    return _mm2(a0, a1, W2, b2[None, :], degT)


def kernel(x, edge_index, W1, b1, W2, b2):
    return _run(x, edge_index, W1, b1, W2, b2)


# speedup vs baseline: 20.4709x; 1.0037x over previous
import functools
import jax
import jax.numpy as jnp
from jax import lax
from jax.experimental import pallas as pl
from jax.experimental.pallas import tpu as pltpu
from jax.experimental.pallas import tpu_sc as plsc

N = 10000
NPAD = 10240
E = 160000
D = 256
H = 128
NS = 16
NC = 2
K = 125
NCH = (E // NS) // K
GRP = 5
NG = NCH // (2 * GRP)
BLK_M = 2000


def _deg_scatter(idx_v, ones_v, shared, sem):
    @pl.loop(0, NCH // GRP)
    def _(g):
        for j in range(GRP):
            pltpu.async_copy(
                ones_v.at[pl.ds(0, K)],
                shared.at[idx_v.at[g, j]],
                sem, add=True)
        for j in range(GRP):
            pltpu.make_async_copy(
                ones_v.at[pl.ds(0, K)],
                shared.at[idx_v.at[g, j]],
                sem).wait()


def _deg_body(rowi, coli, deg_out, idx_v, ones_v, zbuf, shared, sem):
    c = lax.axis_index("c")
    s = lax.axis_index("s")

    @pl.when(c == 0)
    def _():
        pltpu.sync_copy(rowi.at[s], idx_v)

    @pl.when(c == 1)
    def _():
        pltpu.sync_copy(coli.at[s], idx_v)

    one = jnp.ones((16,), jnp.float32)
    for j in range(8):
        ones_v[pl.ds(j * 16, 16)] = one
    zero = jnp.zeros((16,), jnp.float32)

    @pl.loop(0, 40)
    def _(i):
        zbuf[pl.ds(i * 16, 16)] = zero

    pltpu.sync_copy(zbuf, shared.at[pl.ds(s * 640, 640)])
    plsc.subcore_barrier()
    _deg_scatter(idx_v, ones_v, shared, sem)
    plsc.subcore_barrier()
    pltpu.sync_copy(shared.at[pl.ds(s * 640, 640)],
                    deg_out.at[pl.ds(c * NPAD + s * 640, 640)])


@functools.partial(
    pl.kernel,
    out_type=jax.ShapeDtypeStruct((NC * NPAD,), jnp.float32),
    mesh=plsc.VectorSubcoreMesh(core_axis_name="c", subcore_axis_name="s"),
    scratch_types=[
        pltpu.VMEM((NCH // GRP, GRP, K), jnp.int32),
        pltpu.VMEM((128,), jnp.float32),
        pltpu.VMEM((640,), jnp.float32),
        pltpu.VMEM_SHARED((NPAD,), jnp.float32),
        pltpu.SemaphoreType.DMA,
    ],
)
def _deg_kernel(rowi, coli, deg_out, idx_v, ones_v, zbuf, shared, sem):
    _deg_body(rowi, coli, deg_out, idx_v, ones_v, zbuf, shared, sem)


def _agg_group(h_ref, shared, buf, sems, rows_cur, cols_cur, rows_next,
               par0, guard_last):
    for jj in range(GRP):
        bp = (par0 + jj) % 2
        nparity = (par0 + jj + 1) % 2
        if jj < GRP - 1:
            pltpu.async_copy(h_ref.at[rows_cur.at[jj + 1]],
                             buf.at[nparity], sems[nparity])
        elif guard_last is None:
            pltpu.async_copy(h_ref.at[rows_next.at[0]],
                             buf.at[nparity], sems[nparity])
        else:
            @pl.when(guard_last)
            def _():
                pltpu.async_copy(h_ref.at[rows_next.at[0]],
                                 buf.at[nparity], sems[nparity])
        pltpu.make_async_copy(h_ref.at[rows_cur.at[jj]],
                              buf.at[bp], sems[bp]).wait()
        pltpu.sync_copy(buf.at[bp], shared.at[cols_cur.at[jj]], add=True)


def _agg_half(s, h_ref, o_ref, rowi, coli, ri0, ri1, ci0, ci1,
              buf, shared, sem0, sem1):
    sems = (sem0, sem1)
    pltpu.sync_copy(rowi.at[s, 0], ri0)
    pltpu.sync_copy(coli.at[s, 0], ci0)
    pltpu.async_copy(h_ref.at[ri0.at[0]], buf.at[0], sem0)

    @pl.loop(0, NG)
    def _(g):
        pltpu.sync_copy(rowi.at[s, 2 * g + 1], ri1)
        pltpu.sync_copy(coli.at[s, 2 * g + 1], ci1)
        _agg_group(h_ref, shared, buf, sems, ri0, ci0, ri1, 0, None)
        nxt = jnp.minimum(2 * g + 2, 2 * NG - 1)

        @pl.when(g < NG - 1)
        def _():
            pltpu.sync_copy(rowi.at[s, nxt], ri0)
            pltpu.sync_copy(coli.at[s, nxt], ci0)

        _agg_group(h_ref, shared, buf, sems, ri1, ci1, ri0, GRP % 2,
                   g < NG - 1)

    plsc.subcore_barrier()
    for k in range(5):
        pltpu.sync_copy(shared.at[pl.ds(s * 640 + k * 128, 128)],
                        o_ref.at[pl.ds(s * 640 + k * 128, 128)])


def _agg_body(h2, rowi, coli, o0, o1,
              ri0, ri1, ci0, ci1, buf, shared, sem0, sem1):
    c = lax.axis_index("c")
    s = lax.axis_index("s")

    zero = jnp.zeros((16,), jnp.float32)

    @pl.loop(0, 512)
    def _(i):
        buf[0, i // 8, pl.ds((i % 8) * 16, 16)] = zero

    for k in range(10):
        pltpu.sync_copy(buf.at[0, pl.ds(0, 64)],
                        shared.at[pl.ds(s * 640 + k * 64, 64)])
    plsc.subcore_barrier()

    @pl.when(c == 0)
    def _():
        _agg_half(s, h2.at[0], o0, rowi, coli, ri0, ri1, ci0, ci1,
                  buf, shared, sem0, sem1)

    @pl.when(c == 1)
    def _():
        _agg_half(s, h2.at[1], o1, rowi, coli, ri0, ri1, ci0, ci1,
                  buf, shared, sem0, sem1)


@functools.partial(
    pl.kernel,
    out_type=(jax.ShapeDtypeStruct((NPAD, H), jnp.float32),
              jax.ShapeDtypeStruct((NPAD, H), jnp.float32)),
    mesh=plsc.VectorSubcoreMesh(core_axis_name="c", subcore_axis_name="s"),
    scratch_types=[
        pltpu.VMEM((GRP, K), jnp.int32),
        pltpu.VMEM((GRP, K), jnp.int32),
        pltpu.VMEM((GRP, K), jnp.int32),
        pltpu.VMEM((GRP, K), jnp.int32),
        pltpu.VMEM((2, K, H), jnp.float32),
        pltpu.VMEM_SHARED((NPAD, H), jnp.float32),
        pltpu.SemaphoreType.DMA,
        pltpu.SemaphoreType.DMA,
    ],
)
def _agg_kernel(h2, rowi, coli, o0, o1,
                ri0, ri1, ci0, ci1, buf, shared, sem0, sem1):
    _agg_body(h2, rowi, coli, o0, o1,
              ri0, ri1, ci0, ci1, buf, shared, sem0, sem1)


def _mm1_body(x_ref, w1_ref, b1_ref, deg_ref, h_ref):
    xb = x_ref[...].astype(jnp.bfloat16)
    wb = w1_ref[...].astype(jnp.bfloat16)
    acc = lax.dot_general(xb, wb, (((1,), (1,)), ((), ())),
                          preferred_element_type=jnp.float32)
    h = jnp.maximum(acc + b1_ref[...], 0.0)
    d = deg_ref[...]
    dinv = lax.rsqrt(jnp.maximum(d[:, 0:1] + d[:, 1:2], 1.0))
    h = h * dinv
    h_ref[0] = h[:, :H]
    h_ref[1] = h[:, H:]


def _mm1(x, W1, b1, degT):
    return pl.pallas_call(
        _mm1_body,
        grid=(N // BLK_M,),
        in_specs=[
            pl.BlockSpec((BLK_M, D), lambda i: (i, 0)),
            pl.BlockSpec((D, D), lambda i: (0, 0)),
            pl.BlockSpec((1, D), lambda i: (0, 0)),
            pl.BlockSpec((BLK_M, 2), lambda i: (i, 0)),
        ],
        out_specs=pl.BlockSpec((NC, BLK_M, H), lambda i: (0, i, 0)),
        out_shape=jax.ShapeDtypeStruct((NC, N, H), jnp.float32),
    )(x, W1, b1, degT)


def _mm2_body(a0_ref, a1_ref, w2a_ref, w2b_ref, b2_ref, deg_ref, out_ref):
    d = deg_ref[...]
    dinv = lax.rsqrt(jnp.maximum(d[:, 0:1] + d[:, 1:2], 1.0))
    acc = lax.dot_general(a0_ref[...].astype(jnp.bfloat16),
                          w2a_ref[...].astype(jnp.bfloat16),
                          (((1,), (1,)), ((), ())),
                          preferred_element_type=jnp.float32)
    acc = acc + lax.dot_general(a1_ref[...].astype(jnp.bfloat16),
                                w2b_ref[...].astype(jnp.bfloat16),
                                (((1,), (1,)), ((), ())),
                                preferred_element_type=jnp.float32)
    out_ref[...] = acc * dinv + b2_ref[...]


def _mm2(a0, a1, W2, b2, degT):
    return pl.pallas_call(
        _mm2_body,
        grid=(N // BLK_M,),
        in_specs=[
            pl.BlockSpec((BLK_M, H), lambda i: (i, 0)),
            pl.BlockSpec((BLK_M, H), lambda i: (i, 0)),
            pl.BlockSpec((D, H), lambda i: (0, 0)),
            pl.BlockSpec((D, H), lambda i: (0, 1)),
            pl.BlockSpec((1, D), lambda i: (0, 0)),
            pl.BlockSpec((BLK_M, 2), lambda i: (i, 0)),
        ],
        out_specs=pl.BlockSpec((BLK_M, D), lambda i: (i, 0)),
        out_shape=jax.ShapeDtypeStruct((N, D), jnp.float32),
    )(a0, a1, W2, W2, b2, degT)


@jax.jit
def _run(x, edge_index, W1, b1, W2, b2):
    edge32 = edge_index.astype(jnp.int32)
    row, col = edge32[0], edge32[1]
    rowi = row.reshape(NS, NCH // GRP, GRP, K)
    coli = col.reshape(NS, NCH // GRP, GRP, K)

    deg2 = _deg_kernel(rowi, coli)
    degT = deg2.reshape(NC, NPAD).T

    h2 = _mm1(x, W1, b1[None, :], degT)
    a0, a1 = _agg_kernel(h2, rowi, coli)
    return _mm2(a0, a1, W2, b2[None, :], degT)


def kernel(x, edge_index, W1, b1, W2, b2):
    return _run(x, edge_index, W1, b1, W2, b2)
